# baseline (speedup 1.0000x reference)
import functools

import jax
import jax.numpy as jnp
from jax import lax
from jax.experimental import pallas as pl
from jax.experimental.pallas import tpu as pltpu
from jax.experimental.pallas import tpu_sc as plsc

_N_FIELDS = 26
_VOCAB = 100000
_EMBED_DIM = 32
_BATCH = 16384

_NC = 2
_NS = 16
_NW = _NC * _NS
_LANES = 16

_N_OUT_ROWS = _N_FIELDS * _EMBED_DIM
_ROWS_PW = _N_OUT_ROWS // _NW
_Q = _BATCH // 128


def _sc_gather(idx3, tab1):
    mesh = plsc.VectorSubcoreMesh(core_axis_name="c", subcore_axis_name="s")

    @functools.partial(
        pl.kernel,
        mesh=mesh,
        out_type=jax.ShapeDtypeStruct((_N_OUT_ROWS, _BATCH), jnp.float32),
        scratch_types=[
            pltpu.VMEM((_Q, 128), jnp.int32),
            pltpu.VMEM((2, _Q, 128), jnp.int32),
            pltpu.VMEM((2, _BATCH), jnp.float32),
            pltpu.SemaphoreType.DMA,
            pltpu.SemaphoreType.DMA,
        ],
    )
    def k(idx_hbm, tab_hbm, out_hbm, idxv, eidx, rows, gs0, gs1):
        wid = lax.axis_index("s") * _NC + lax.axis_index("c")
        gsems = [gs0, gs1]
        j0 = wid * _ROWS_PW

        def fire(r, b):
            j = j0 + r
            pltpu.sync_copy(idx_hbm.at[lax.div(j, _EMBED_DIM)], idxv)
            base = j * _VOCAB

            def stream(q, carry):
                for l in range(128 // _LANES):
                    sl = pl.ds(l * _LANES, _LANES)
                    eidx[b, q, sl] = idxv[q, sl] + base
                pltpu.async_copy(tab_hbm.at[eidx.at[b, q]],
                                 rows.at[b, pl.ds(q * 128, 128)], gsems[b])
                return carry

            lax.fori_loop(0, _Q, stream, 0, unroll=False)

        def finish(r, b):
            pltpu.make_async_copy(out_hbm.at[0], rows.at[b],
                                  gsems[b]).wait()
            pltpu.sync_copy(rows.at[b], out_hbm.at[j0 + r])

        fire(0, 0)

        def pair(rr, carry):
            for d in range(2):
                r = rr + d
                fire(r + 1, 1 - d)
                finish(r, d)
            return carry

        lax.fori_loop(0, (_ROWS_PW - 2) // 2, lambda i, u: pair(2 * i, u),
                      0, unroll=False)
        fire(_ROWS_PW - 1, 1)
        finish(_ROWS_PW - 2, 0)
        finish(_ROWS_PW - 1, 1)

    return k(idx3, tab1)


def kernel(inputs, tables):
    idx3 = inputs.astype(jnp.int32).T.reshape(_N_FIELDS, _Q, 128)
    tab1 = jnp.transpose(tables, (0, 2, 1)).reshape(-1)
    out = _sc_gather(idx3, tab1)
    return out.T.reshape(_BATCH, _N_FIELDS * _EMBED_DIM)


# speedup vs baseline: 1.3774x; 1.3774x over previous
import functools

import jax
import jax.numpy as jnp
from jax import lax
from jax.experimental import pallas as pl
from jax.experimental.pallas import tpu as pltpu
from jax.experimental.pallas import tpu_sc as plsc

_N_FIELDS = 26
_VOCAB = 100000
_EMBED_DIM = 32
_BATCH = 16384

_NC = 2
_NS = 16
_LANES = 16

_N_OUT_ROWS = _N_FIELDS * _EMBED_DIM
_RT = 8
_BLK = _RT * _VOCAB
_N_ROUNDS = _N_OUT_ROWS // _RT // _NC
_CHUNK = _BLK // _NS
_BPS = _BATCH // _NS
_QS = _BPS // 128


def _sc_gather(idx3, tab1):
    mesh = plsc.VectorSubcoreMesh(core_axis_name="c", subcore_axis_name="s")

    @functools.partial(
        pl.kernel,
        mesh=mesh,
        out_type=jax.ShapeDtypeStruct((_N_OUT_ROWS, _BATCH), jnp.float32),
        scratch_types=[
            pltpu.VMEM((_QS, 128), jnp.int32),
            pltpu.VMEM((_RT, _QS, 128), jnp.int32),
            pltpu.VMEM((_RT, _BPS), jnp.float32),
            pltpu.VMEM_SHARED((2 * _BLK,), jnp.float32),
            pltpu.SemaphoreType.DMA,
            pltpu.SemaphoreType.DMA,
        ],
    )
    def k(idx_hbm, tab_hbm, out_hbm, idxv, eidx, rows_v, sh, gsem, ssem):
        c = lax.axis_index("c")
        s = lax.axis_index("s")
        rt0 = _N_ROUNDS * c

        _PCH = 6400
        _NPC = _BLK // _PCH
        _KMAX = (_NPC + _NS - 1) // _NS

        def stage(m, buf):
            for kk in range(_KMAX):
                p = kk * _NS + s

                @pl.when(p < _NPC)
                def _():
                    pltpu.async_copy(
                        tab_hbm.at[pl.ds((rt0 + m) * _BLK + p * _PCH,
                                         _PCH)],
                        sh.at[pl.ds(buf * _BLK + p * _PCH, _PCH)], ssem)

        def drain_stage(buf):
            for kk in range(_KMAX):
                p = kk * _NS + s

                @pl.when(p < _NPC)
                def _():
                    pltpu.make_async_copy(
                        tab_hbm.at[pl.ds(0, _PCH)],
                        sh.at[pl.ds(buf * _BLK + p * _PCH, _PCH)],
                        ssem).wait()

        def do_round(m, buf):
            rt = rt0 + m
            f = lax.div(rt * _RT, _EMBED_DIM)
            pltpu.sync_copy(idx_hbm.at[f, pl.ds(_QS * s, _QS)], idxv)
            sbase = buf * _BLK

            def fire_row(ld, c2):
                base = sbase + ld * _VOCAB
                for q in range(_QS):
                    for l in range(128 // _LANES):
                        sl = pl.ds(l * _LANES, _LANES)
                        eidx[ld, q, sl] = idxv[q, sl] + base
                    pltpu.async_copy(sh.at[eidx.at[ld, q]],
                                     rows_v.at[ld, pl.ds(q * 128, 128)],
                                     gsem)
                return c2

            lax.fori_loop(0, _RT, fire_row, 0, unroll=False)
            pltpu.make_async_copy(
                out_hbm.at[pl.ds(0, _RT), pl.ds(0, _BPS)], rows_v,
                gsem).wait()

            def write_row(ld, c2):
                pltpu.sync_copy(rows_v.at[ld],
                                out_hbm.at[rt * _RT + ld,
                                           pl.ds(_BPS * s, _BPS)])
                return c2

            lax.fori_loop(0, _RT, write_row, 0, unroll=False)

        stage(0, 0)
        drain_stage(0)
        plsc.subcore_barrier()

        def pair(mm, carry):
            for d in range(2):
                m = mm + d
                buf = d
                @pl.when(m + 1 < _N_ROUNDS)
                def _():
                    stage(m + 1, 1 - buf)
                do_round(m, buf)
                @pl.when(m + 1 < _N_ROUNDS)
                def _():
                    drain_stage(1 - buf)
                plsc.subcore_barrier()
            return carry

        lax.fori_loop(0, _N_ROUNDS // 2, lambda i, u: pair(2 * i, u), 0,
                      unroll=False)

    return k(idx3, tab1)


def kernel(inputs, tables):
    idx3 = inputs.astype(jnp.int32).T.reshape(_N_FIELDS, _BATCH // 128, 128)
    tab1 = jnp.transpose(tables, (0, 2, 1)).reshape(-1)
    out = _sc_gather(idx3, tab1)
    return out.T.reshape(_BATCH, _N_FIELDS * _EMBED_DIM)
